# baseline (speedup 1.0000x reference)
import functools

import jax
import jax.numpy as jnp
import numpy as np
from jax import lax
from jax.experimental import pallas as pl
from jax.experimental.pallas import tpu as pltpu
from jax.experimental.pallas import tpu_sc as plsc

HIDDEN = 64
SEQ = 50
UNIT = 2 * SEQ
NC, NS = 2, 16
NW = NC * NS


def _pos_enc(seq_len: int, ch: int) -> np.ndarray:
    channels = int(np.ceil(ch / 2) * 2)
    inv_freq = 1.0 / (10000 ** (np.arange(0, channels, 2).astype(np.float32) / channels))
    pos = np.arange(seq_len).astype(np.float32)
    sin_inp = np.einsum("i,j->ij", pos, inv_freq)
    emb = np.stack((np.sin(sin_inp), np.cos(sin_inp)), axis=-1).reshape(seq_len, channels)
    return emb[:, :ch].astype(np.float32)


def kernel(x, dummy_sigma, embedding):
    del dummy_sigma
    n_units = (x.shape[0] * x.shape[1]) // UNIT
    units_per_w = n_units // NW
    idx2d = x.reshape(n_units, UNIT)
    pe = np.tile(_pos_enc(SEQ, HIDDEN), (UNIT // SEQ, 1))
    pe2 = jnp.asarray(pe)

    mesh = plsc.VectorSubcoreMesh(core_axis_name="c", subcore_axis_name="s")

    @functools.partial(
        pl.kernel,
        out_type=jax.ShapeDtypeStruct((n_units, UNIT, HIDDEN), jnp.float32),
        mesh=mesh,
        compiler_params=pltpu.CompilerParams(use_tc_tiling_on_sc=False),
        scratch_types=[
            pltpu.VMEM((units_per_w, UNIT), jnp.int32),
            pltpu.VMEM((UNIT, HIDDEN), jnp.float32),
            pltpu.VMEM((UNIT, HIDDEN), jnp.float32),
            pltpu.SemaphoreType.DMA,
        ],
    )
    def sc_kernel(table_hbm, idx_hbm, pe_hbm, out_hbm, idx_v, pe_v, buf, sem):
        wid = lax.axis_index("s") * NC + lax.axis_index("c")
        base = wid * units_per_w
        pltpu.sync_copy(idx_hbm.at[pl.ds(base, units_per_w)], idx_v)
        pltpu.sync_copy(pe_hbm, pe_v)

        def unit_body(j, carry):
            pltpu.async_copy(table_hbm.at[idx_v.at[j]], buf, sem).wait()

            def add_row(i, c):
                for g in range(HIDDEN // 16):
                    sl = pl.ds(g * 16, 16)
                    buf[i, sl] = buf[i, sl] + pe_v[i, sl]
                return c

            lax.fori_loop(0, UNIT, add_row, 0)
            pltpu.sync_copy(buf, out_hbm.at[base + j])
            return carry

        lax.fori_loop(0, units_per_w, unit_body, 0)

    out = sc_kernel(embedding, idx2d, pe2)
    return out.reshape(x.shape[0], x.shape[1], HIDDEN)


# speedup vs baseline: 1.0013x; 1.0013x over previous
import functools

import jax
import jax.numpy as jnp
import numpy as np
from jax import lax
from jax.experimental import pallas as pl
from jax.experimental.pallas import tpu as pltpu
from jax.experimental.pallas import tpu_sc as plsc

HIDDEN = 64
SEQ = 50
UNIT = 2 * SEQ
NC, NS = 2, 16
NW = NC * NS
NBUF = 4


def _pos_enc(seq_len: int, ch: int) -> np.ndarray:
    channels = int(np.ceil(ch / 2) * 2)
    inv_freq = 1.0 / (10000 ** (np.arange(0, channels, 2).astype(np.float32) / channels))
    pos = np.arange(seq_len).astype(np.float32)
    sin_inp = np.einsum("i,j->ij", pos, inv_freq)
    emb = np.stack((np.sin(sin_inp), np.cos(sin_inp)), axis=-1).reshape(seq_len, channels)
    return emb[:, :ch].astype(np.float32)


def kernel(x, dummy_sigma, embedding):
    del dummy_sigma
    n_units = (x.shape[0] * x.shape[1]) // UNIT
    units_per_w = n_units // NW
    idx2d = x.reshape(n_units, UNIT)
    pe2 = jnp.asarray(np.tile(_pos_enc(SEQ, HIDDEN), (UNIT // SEQ, 1)))

    mesh = plsc.VectorSubcoreMesh(core_axis_name="c", subcore_axis_name="s")

    @functools.partial(
        pl.kernel,
        out_type=jax.ShapeDtypeStruct((n_units, UNIT, HIDDEN), jnp.float32),
        mesh=mesh,
        compiler_params=pltpu.CompilerParams(use_tc_tiling_on_sc=False),
        scratch_types=[
            pltpu.VMEM((units_per_w, UNIT), jnp.int32),
            pltpu.VMEM((UNIT, HIDDEN), jnp.float32),
            pltpu.VMEM((NBUF, UNIT, HIDDEN), jnp.float32),
            pltpu.VMEM((NBUF, UNIT, HIDDEN), jnp.float32),
            pltpu.SemaphoreType.DMA((NBUF,)),
            pltpu.SemaphoreType.DMA((NBUF,)),
        ],
    )
    def sc_kernel(table_hbm, idx_hbm, pe_hbm, out_hbm,
                  idx_v, pe_v, ibuf, obuf, gsem, ssem):
        wid = lax.axis_index("s") * NC + lax.axis_index("c")
        base = wid * units_per_w
        pltpu.sync_copy(idx_hbm.at[pl.ds(base, units_per_w)], idx_v)
        pltpu.sync_copy(pe_hbm, pe_v)

        def gather(j, b):
            return pltpu.make_async_copy(
                table_hbm.at[idx_v.at[j]], ibuf.at[b], gsem.at[b])

        def store(j, b):
            return pltpu.make_async_copy(
                obuf.at[b], out_hbm.at[base + j], ssem.at[b])

        def add_pe(b):
            def add_row(i, c):
                for g in range(HIDDEN // 16):
                    sl = pl.ds(g * 16, 16)
                    obuf[b, i, sl] = ibuf[b, i, sl] + pe_v[i, sl]
                return c
            lax.fori_loop(0, UNIT, add_row, 0, unroll=2)

        for b in range(NBUF):
            gather(b, b).start()

        def main_step(g, carry):
            for b in range(NBUF):
                j = g * NBUF + b
                gather(j, b).wait()

                @pl.when(j >= NBUF)
                def _():
                    store(j - NBUF, b).wait()

                add_pe(b)
                gather(j + NBUF, b).start()
                store(j, b).start()
            return carry

        lax.fori_loop(0, (units_per_w - NBUF) // NBUF, main_step, 0,
                      unroll=1)

        for b in range(NBUF):
            j = units_per_w - NBUF + b
            gather(j, b).wait()
            store(j - NBUF, b).wait()
            add_pe(b)
            store(j, b).start()
        for b in range(NBUF):
            store(units_per_w - NBUF + b, b).wait()

    out = sc_kernel(embedding, idx2d, pe2)
    return out.reshape(x.shape[0], x.shape[1], HIDDEN)
